# baseline (speedup 1.0000x reference)
import functools

import jax
import jax.numpy as jnp
from jax import lax
from jax.experimental import pallas as pl
from jax.experimental.pallas import tpu as pltpu
from jax.experimental.pallas import tpu_sc as plsc

NUM_CORES = 2
NUM_SUBCORES = 16
NUM_TILES = NUM_CORES * NUM_SUBCORES
CHUNK = 80
REL_PAD = 480


def _sc_aggregate(x, idx3, rel_emb, zero_d, onehot, num_chunks):
    n, d = x.shape
    chunks_per_tile = num_chunks // NUM_TILES
    assert chunks_per_tile * NUM_TILES == num_chunks
    assert chunks_per_tile % 2 == 1
    pairs = (chunks_per_tile - 1) // 2
    rows_per_tile = -(-n // (NUM_SUBCORES * CHUNK)) * CHUNK
    n_pad = rows_per_tile * NUM_SUBCORES
    stages = rows_per_tile // CHUNK

    mesh = plsc.VectorSubcoreMesh(
        core_axis_name="c",
        subcore_axis_name="s",
        num_cores=NUM_CORES,
        num_subcores=NUM_SUBCORES,
    )

    @functools.partial(
        pl.kernel,
        out_type=(
            jax.ShapeDtypeStruct((NUM_CORES, n_pad, d), jnp.float32),
            jax.ShapeDtypeStruct((NUM_CORES, n_pad, d), jnp.float32),
        ),
        mesh=mesh,
        scratch_types=[
            pltpu.VMEM_SHARED((n_pad, d), jnp.float32),
            pltpu.VMEM((3, CHUNK), jnp.int32),
            pltpu.VMEM((3, CHUNK), jnp.int32),
            pltpu.VMEM((CHUNK, d), jnp.float32),
            pltpu.VMEM((CHUNK, d), jnp.float32),
            pltpu.VMEM((CHUNK, d), jnp.float32),
            pltpu.VMEM((CHUNK, d), jnp.float32),
            pltpu.SemaphoreType.DMA,
            pltpu.SemaphoreType.DMA,
            pltpu.SemaphoreType.DMA,
            pltpu.SemaphoreType.DMA,
            pltpu.SemaphoreType.DMA,
            pltpu.SemaphoreType.DMA,
            pltpu.SemaphoreType.DMA,
            pltpu.SemaphoreType.DMA,
        ],
    )
    def sc_kernel(
        x_hbm, idx3_hbm, rel_hbm, zd_hbm, oh_hbm,
        out_hbm, outdeg_hbm,
        accum,
        idx0, idx1, xb0, xb1, rb0, rb1,
        semi0, semi1, semg0, semg1, sems0, sems1, semw0, semw1,
    ):
        cid = lax.axis_index("c")
        sid = lax.axis_index("s")
        wid = cid * NUM_SUBCORES + sid
        row0 = sid * rows_per_tile
        idxs = (idx0, idx1)
        xbs = (xb0, xb1)
        rbs = (rb0, rb1)
        semis = (semi0, semi1)
        semgs = (semg0, semg1)
        semws = (semw0, semw1)
        bufs = (xb0, xb1)

        def chunk_of(k):
            return wid + k * NUM_TILES

        def load_idx(k, b):
            return pltpu.async_copy(idx3_hbm.at[chunk_of(k)], idxs[b],
                                    semis[b])

        def start_gathers(b):
            pltpu.async_copy(x_hbm.at[idxs[b].at[0]], xbs[b], semgs[b])
            return pltpu.async_copy(rel_hbm.at[idxs[b].at[1]], rbs[b],
                                    semgs[b])

        def wait_idx(b):
            pltpu.make_async_copy(idx3_hbm.at[0], idxs[b], semis[b]).wait()

        def wait_gathers(b):
            pltpu.make_async_copy(x_hbm.at[idxs[b].at[0]], xbs[b],
                                  semgs[b]).wait()
            pltpu.make_async_copy(rel_hbm.at[idxs[b].at[1]], rbs[b],
                                  semgs[b]).wait()

        def scatter(b):
            cpx = pltpu.async_copy(xbs[b], accum.at[idxs[b].at[2]], sems0,
                                   add=True)
            cpr = pltpu.async_copy(rbs[b], accum.at[idxs[b].at[2]], sems1,
                                   add=True)
            cpx.wait()
            cpr.wait()

        pltpu.sync_copy(zd_hbm, xb0)

        for j in range(stages):
            pltpu.sync_copy(xb0, accum.at[pl.ds(row0 + j * CHUNK, CHUNK), :])

        plsc.subcore_barrier()

        load_idx(0, 0).wait()
        g0 = start_gathers(0)
        load_idx(1, 1)

        @pl.loop(0, pairs)
        def _(t):
            k = 2 * t
            wait_gathers(0)
            wait_idx(1)
            start_gathers(1)
            scatter(0)
            load_idx(k + 2, 0)
            wait_gathers(1)
            wait_idx(0)
            start_gathers(0)
            scatter(1)
            load_idx(k + 3, 1)

        wait_gathers(0)
        wait_idx(1)
        scatter(0)

        plsc.subcore_barrier()

        def write_out(dst_hbm3):
            for j in range(stages):
                b = j & 1
                r = row0 + j * CHUNK
                if j >= 2:
                    pltpu.make_async_copy(
                        bufs[b], dst_hbm3.at[cid, pl.ds(row0, CHUNK), :],
                        semws[b],
                    ).wait()
                pltpu.sync_copy(accum.at[pl.ds(r, CHUNK), :], bufs[b])
                pltpu.async_copy(
                    bufs[b], dst_hbm3.at[cid, pl.ds(r, CHUNK), :], semws[b]
                )
            for b in range(2):
                pltpu.make_async_copy(
                    bufs[b], dst_hbm3.at[cid, pl.ds(row0, CHUNK), :], semws[b]
                ).wait()

        write_out(out_hbm)
        plsc.subcore_barrier()

        pltpu.sync_copy(oh_hbm, rb0)
        load_idx(0, 0).wait()
        load_idx(1, 1)

        @pl.loop(0, pairs)
        def _(t):
            k = 2 * t
            cp0 = pltpu.async_copy(rb0, accum.at[idxs[0].at[2]], sems0,
                                   add=True)
            wait_idx(1)
            cp1 = pltpu.async_copy(rb0, accum.at[idxs[1].at[2]], sems1,
                                   add=True)
            cp0.wait()
            load_idx(k + 2, 0)
            cp1.wait()
            load_idx(k + 3, 1)
            wait_idx(0)

        pltpu.async_copy(rb0, accum.at[idxs[0].at[2]], sems0,
                         add=True).wait()
        wait_idx(1)

        plsc.subcore_barrier()
        write_out(outdeg_hbm)

    return sc_kernel(x, idx3, rel_emb, zero_d, onehot)


_SLOPE = (1.0 / 8.0 + 1.0 / 3.0) / 2.0


def _tc_combine_body(p0, p1, q0, q1, xb, wn, lw, elw, o):
    acc = p0[...] + p1[...]
    aft = q0[:, 0] + q1[:, 0]
    deg = jnp.round(aft - acc[:, 0])
    prec = lax.Precision.HIGHEST
    h = lax.dot(acc, wn[...], precision=prec)
    norm = 1.0 / jnp.maximum(deg, 1.0)
    loop_main = lax.dot(xb[...], lw[...], precision=prec)
    loop_evolve = lax.dot(xb[...], elw[...], precision=prec)
    loop_msg = jnp.where((deg > 0.0)[:, None], loop_main, loop_evolve)
    y = h * norm[:, None] + loop_msg
    o[...] = jnp.where(y >= 0.0, y, y * _SLOPE)


def _tc_combine(parts, degparts, x, wn, lw, elw):
    n, d = x.shape
    blk = 1000
    grid = n // blk
    assert grid * blk == n
    row_spec = pl.BlockSpec((blk, d), lambda i: (i, 0))
    full_spec = pl.BlockSpec((d, d), lambda i: (0, 0))
    return pl.pallas_call(
        _tc_combine_body,
        grid=(grid,),
        in_specs=[row_spec, row_spec, row_spec, row_spec, row_spec,
                  full_spec, full_spec, full_spec],
        out_specs=row_spec,
        out_shape=jax.ShapeDtypeStruct((n, d), jnp.float32),
    )(parts[0], parts[1], degparts[0], degparts[1], x, wn, lw, elw)


def kernel(x, edge_index, edge_rel, rel_emb, W_neighbor, loop_weight,
           evolve_loop_weight):
    n, d = x.shape
    e = edge_index.shape[1]
    num_chunks = e // CHUNK
    assert num_chunks * CHUNK == e
    idx3 = jnp.stack(
        [
            edge_index[0].reshape(num_chunks, CHUNK),
            edge_rel.reshape(num_chunks, CHUNK),
            edge_index[1].reshape(num_chunks, CHUNK),
        ],
        axis=1,
    )
    idx3 = jnp.concatenate(
        [idx3, jnp.zeros((2 * NUM_TILES, 3, CHUNK), jnp.int32)], axis=0
    )
    rel_padded = jnp.zeros((REL_PAD, d), jnp.float32).at[
        : rel_emb.shape[0]
    ].set(rel_emb)
    zero_d = jnp.zeros((CHUNK, d), jnp.float32)
    onehot = zero_d.at[:, 0].set(1.0)
    parts, degparts = _sc_aggregate(
        x, idx3, rel_padded, zero_d, onehot, num_chunks
    )
    return _tc_combine(parts, degparts, x, W_neighbor, loop_weight,
                       evolve_loop_weight)


# speedup vs baseline: 1.0220x; 1.0220x over previous
import dataclasses
import functools

import jax
import jax.numpy as jnp
from jax import lax
from jax.experimental import pallas as pl
from jax.experimental.pallas import tpu as pltpu
from jax.experimental.pallas import tpu_sc as plsc

NUM_CORES = 2
NUM_SUBCORES = 16
NUM_TILES = NUM_CORES * NUM_SUBCORES
CHUNK = 64
LANES = 16
REL_PAD = 512


def _sc_aggregate(x, idx3, rel_emb, zero_d, num_chunks):
    n, d = x.shape
    chunks_per_tile = num_chunks // NUM_TILES
    assert chunks_per_tile * NUM_TILES == num_chunks
    assert chunks_per_tile % 2 == 1
    pairs = (chunks_per_tile - 1) // 2
    rows_per_tile = -(-n // (NUM_SUBCORES * CHUNK)) * CHUNK
    n_pad = rows_per_tile * NUM_SUBCORES
    stages = rows_per_tile // CHUNK

    mesh = plsc.VectorSubcoreMesh(
        core_axis_name="c",
        subcore_axis_name="s",
        num_cores=NUM_CORES,
        num_subcores=NUM_SUBCORES,
    )

    cp = pltpu.CompilerParams()
    if "needs_layout_passes" in pltpu.CompilerParams.__dataclass_fields__:
        cp = dataclasses.replace(cp, needs_layout_passes=False)

    @functools.partial(
        pl.kernel,
        compiler_params=cp,
        out_type=(
            jax.ShapeDtypeStruct((NUM_CORES, n_pad, d), jnp.float32),
            jax.ShapeDtypeStruct((NUM_TILES * n_pad,), jnp.float32),
        ),
        mesh=mesh,
        scratch_types=[
            pltpu.VMEM_SHARED((n_pad, d), jnp.float32),
            pltpu.VMEM((3, CHUNK), jnp.int32),
            pltpu.VMEM((3, CHUNK), jnp.int32),
            pltpu.VMEM((CHUNK, d), jnp.float32),
            pltpu.VMEM((CHUNK, d), jnp.float32),
            pltpu.VMEM((CHUNK, d), jnp.float32),
            pltpu.VMEM((CHUNK, d), jnp.float32),
            pltpu.VMEM((n_pad,), jnp.float32),
            pltpu.SemaphoreType.DMA,
            pltpu.SemaphoreType.DMA,
            pltpu.SemaphoreType.DMA,
            pltpu.SemaphoreType.DMA,
            pltpu.SemaphoreType.DMA,
            pltpu.SemaphoreType.DMA,
            pltpu.SemaphoreType.DMA,
            pltpu.SemaphoreType.DMA,
        ],
    )
    def sc_kernel(
        x_hbm, idx3_hbm, rel_hbm, zd_hbm,
        out_hbm, outdeg_hbm,
        accum,
        idx0, idx1, xb0, xb1, rb0, rb1, degtile,
        semi0, semi1, semg0, semg1, sems0, sems1, semw0, semw1,
    ):
        cid = lax.axis_index("c")
        sid = lax.axis_index("s")
        wid = cid * NUM_SUBCORES + sid
        row0 = sid * rows_per_tile
        idxs = (idx0, idx1)
        xbs = (xb0, xb1)
        rbs = (rb0, rb1)
        semis = (semi0, semi1)
        semgs = (semg0, semg1)
        semws = (semw0, semw1)
        ones16 = jnp.full((LANES,), 1.0, jnp.float32)
        zeros16 = jnp.zeros((LANES,), jnp.float32)

        def chunk_of(k):
            return wid + k * NUM_TILES

        def load_idx(k, b):
            return pltpu.async_copy(idx3_hbm.at[chunk_of(k)], idxs[b],
                                    semis[b])

        def start_gathers(b):
            pltpu.async_copy(x_hbm.at[idxs[b].at[0]], xbs[b], semgs[b])
            pltpu.async_copy(rel_hbm.at[idxs[b].at[1]], rbs[b], semgs[b])

        def wait_idx(b):
            pltpu.make_async_copy(idx3_hbm.at[0], idxs[b], semis[b]).wait()

        def wait_gathers(b):
            pltpu.make_async_copy(x_hbm.at[idxs[b].at[0]], xbs[b],
                                  semgs[b]).wait()
            pltpu.make_async_copy(rel_hbm.at[idxs[b].at[1]], rbs[b],
                                  semgs[b]).wait()

        def scatter(b):
            cpx = pltpu.async_copy(xbs[b], accum.at[idxs[b].at[2]], sems0,
                                   add=True)
            cpr = pltpu.async_copy(rbs[b], accum.at[idxs[b].at[2]], sems1,
                                   add=True)
            for g in range(CHUNK // LANES):
                dstv = idxs[b][2, pl.ds(g * LANES, LANES)]
                plsc.addupdate_scatter(degtile, [dstv], ones16)
            cpx.wait()
            cpr.wait()

        pltpu.sync_copy(zd_hbm, xb0)

        for j in range(stages):
            pltpu.sync_copy(xb0, accum.at[pl.ds(row0 + j * CHUNK, CHUNK), :])

        @pl.loop(0, n_pad // LANES)
        def _(i):
            degtile[pl.ds(i * LANES, LANES)] = zeros16

        plsc.subcore_barrier()

        load_idx(0, 0).wait()
        start_gathers(0)
        load_idx(1, 1)

        @pl.loop(0, pairs)
        def _(t):
            k = 2 * t
            wait_gathers(0)
            wait_idx(1)
            start_gathers(1)
            scatter(0)
            load_idx(k + 2, 0)
            wait_gathers(1)
            wait_idx(0)
            start_gathers(0)
            scatter(1)
            load_idx(k + 3, 1)

        wait_gathers(0)
        wait_idx(1)
        scatter(0)

        pltpu.sync_copy(degtile, outdeg_hbm.at[pl.ds(wid * n_pad, n_pad)])

        plsc.subcore_barrier()

        for j in range(stages):
            b = j & 1
            r = row0 + j * CHUNK
            if j >= 2:
                pltpu.make_async_copy(
                    xbs[b], out_hbm.at[cid, pl.ds(row0, CHUNK), :], semws[b]
                ).wait()
            pltpu.sync_copy(accum.at[pl.ds(r, CHUNK), :], xbs[b])
            pltpu.async_copy(
                xbs[b], out_hbm.at[cid, pl.ds(r, CHUNK), :], semws[b]
            )
        for b in range(2):
            pltpu.make_async_copy(
                xbs[b], out_hbm.at[cid, pl.ds(row0, CHUNK), :], semws[b]
            ).wait()

    return sc_kernel(x, idx3, rel_emb, zero_d)


_SLOPE = (1.0 / 8.0 + 1.0 / 3.0) / 2.0


def _tc_combine_body(p0, p1, dall, xb, wn, lw, elw, o):
    acc = p0[...] + p1[...]
    deg = jnp.sum(dall[...], axis=1, keepdims=True)
    prec = lax.Precision.HIGHEST
    h = lax.dot(acc, wn[...], precision=prec)
    norm = 1.0 / jnp.maximum(deg, 1.0)
    loop_main = lax.dot(xb[...], lw[...], precision=prec)
    loop_evolve = lax.dot(xb[...], elw[...], precision=prec)
    loop_msg = jnp.where(deg > 0.0, loop_main, loop_evolve)
    y = h * norm + loop_msg
    o[...] = jnp.where(y >= 0.0, y, y * _SLOPE)


def _tc_combine(parts, degparts, x_pad, wn, lw, elw):
    n_pad, d = x_pad.shape
    blk = 1280
    grid = n_pad // blk
    assert grid * blk == n_pad
    row_spec = pl.BlockSpec((blk, d), lambda i: (i, 0))
    deg_spec = pl.BlockSpec((blk, NUM_TILES), lambda i: (i, 0))
    full_spec = pl.BlockSpec((d, d), lambda i: (0, 0))
    return pl.pallas_call(
        _tc_combine_body,
        grid=(grid,),
        in_specs=[row_spec, row_spec, deg_spec, row_spec,
                  full_spec, full_spec, full_spec],
        out_specs=row_spec,
        out_shape=jax.ShapeDtypeStruct((n_pad, d), jnp.float32),
    )(parts[0], parts[1], degparts, x_pad, wn, lw, elw)


def kernel(x, edge_index, edge_rel, rel_emb, W_neighbor, loop_weight,
           evolve_loop_weight):
    n, d = x.shape
    e = edge_index.shape[1]
    rows_per_tile = -(-n // (NUM_SUBCORES * CHUNK)) * CHUNK
    n_pad = rows_per_tile * NUM_SUBCORES
    num_chunks = -(-e // CHUNK)
    cpt = -(-num_chunks // NUM_TILES)
    if cpt % 2 == 0:
        cpt += 1
    num_chunks = cpt * NUM_TILES
    e_pad = num_chunks * CHUNK
    src = jnp.concatenate(
        [edge_index[0], jnp.zeros((e_pad - e,), jnp.int32)]
    )
    rel = jnp.concatenate([edge_rel, jnp.zeros((e_pad - e,), jnp.int32)])
    dst = jnp.concatenate(
        [edge_index[1], jnp.full((e_pad - e,), n_pad - 1, jnp.int32)]
    )
    idx3 = jnp.stack(
        [
            src.reshape(num_chunks, CHUNK),
            rel.reshape(num_chunks, CHUNK),
            dst.reshape(num_chunks, CHUNK),
        ],
        axis=1,
    )
    idx3 = jnp.concatenate(
        [idx3, jnp.zeros((2 * NUM_TILES, 3, CHUNK), jnp.int32)], axis=0
    )
    rel_padded = jnp.zeros((REL_PAD, d), jnp.float32).at[
        : rel_emb.shape[0]
    ].set(rel_emb)
    zero_d = jnp.zeros((CHUNK, d), jnp.float32)
    x_pad = jnp.concatenate(
        [x, jnp.zeros((n_pad - n, d), jnp.float32)], axis=0
    )
    parts, degflat = _sc_aggregate(x, idx3, rel_padded, zero_d, num_chunks)
    degparts = degflat.reshape(NUM_TILES, n_pad).T
    out = _tc_combine(parts, degparts, x_pad, W_neighbor, loop_weight,
                      evolve_loop_weight)
    return out[:n]
